# Initial kernel scaffold; baseline (speedup 1.0000x reference)
import jax
import jax.numpy as jnp
from jax.experimental import pallas as pl


def kernel(x, wsq, wg, b_gwc, masks):
    raise NotImplementedError("write your pallas kernel here")


# speedup vs baseline: 1.0458x; 1.0458x over previous
import functools

import jax
import jax.numpy as jnp
from jax.experimental import pallas as pl
from jax.experimental.pallas import tpu as pltpu


def _cru_body(uq, H, W, kk, x_ref, wsq_ref, wg_ref, bias_ref, mask_ref, o_ref):
    S = H * W
    pad = kk // 2

    x = x_ref[0].astype(jnp.bfloat16)
    wsq = wsq_ref[...]

    ul = jnp.dot(wsq, x, preferred_element_type=jnp.float32)
    u = ul[:uq, :].astype(jnp.bfloat16)
    y2 = ul[uq:, :]

    taps = []
    t = 0
    for ky in range(kk):
        for kx in range(kk):
            dy, dx = ky - pad, kx - pad
            if dy == 0 and dx == 0:
                taps.append(u)
            else:
                shift = (-(dy * W + dx)) % S
                rolled = pltpu.roll(u, shift=shift, axis=1)
                taps.append(rolled * mask_ref[t:t + 1, :])
            t += 1
    ucat = jnp.concatenate(taps, axis=0)

    y1 = jnp.dot(wg_ref[...], ucat,
                 preferred_element_type=jnp.float32) + bias_ref[...]

    m1 = jnp.mean(y1, axis=1, keepdims=True)
    m2 = jnp.mean(y2, axis=1, keepdims=True)
    mx = jnp.maximum(jnp.max(m1), jnp.max(m2))
    e1 = jnp.exp(m1 - mx)
    e2 = jnp.exp(m2 - mx)
    inv = 1.0 / (jnp.sum(e1) + jnp.sum(e2))
    o_ref[0] = (e1 * inv) * y1 + (e2 * inv) * y2


def kernel(x, wsq, wg, b_gwc, masks):
    N, C, H, W = x.shape
    S = H * W
    uq = wsq.shape[0] - C
    n_taps = masks.shape[0]
    kk = int(round(n_taps ** 0.5))
    kq = n_taps * uq

    xr = x.reshape(N, C, S)

    body = functools.partial(_cru_body, uq, H, W, kk)

    est = (2 * C * S * 4 + 2 * C * S * 4 + kq * S * 2 + 4 * C * S * 4
           + n_taps * S * 2 + (uq + C) * C * 2 + C * kq * 2 + C * 4)
    vmem_limit = int(min(max(2 * est, 32 * 1024 * 1024),
                         int(64 * 1024 * 1024 * 0.9)))

    out = pl.pallas_call(
        body,
        out_shape=jax.ShapeDtypeStruct((N, C, S), jnp.float32),
        grid=(N,),
        in_specs=[
            pl.BlockSpec((1, C, S), lambda b: (b, 0, 0)),
            pl.BlockSpec(wsq.shape, lambda b: (0, 0)),
            pl.BlockSpec(wg.shape, lambda b: (0, 0)),
            pl.BlockSpec(b_gwc.shape, lambda b: (0, 0)),
            pl.BlockSpec(masks.shape, lambda b: (0, 0)),
        ],
        out_specs=pl.BlockSpec((1, C, S), lambda b: (b, 0, 0)),
        compiler_params=pltpu.CompilerParams(
            dimension_semantics=("parallel",),
            vmem_limit_bytes=vmem_limit),
    )(xr, wsq, wg, b_gwc, masks)

    return out.reshape(N, C, H, W)


# speedup vs baseline: 1.3180x; 1.2602x over previous
import functools

import jax
import jax.numpy as jnp
from jax.experimental import pallas as pl
from jax.experimental.pallas import tpu as pltpu


def _cru_body(uq, H, W, kk, nblk, x_ref, wsq_ref, wg_ref, bias_ref, mask_ref,
              o_ref):
    S = H * W
    pad = kk // 2
    wsq = wsq_ref[...]
    wg = wg_ref[...]
    bias = bias_ref[...]

    for i in range(nblk):
        x = x_ref[i].astype(jnp.bfloat16)

        ul = jnp.dot(wsq, x, preferred_element_type=jnp.float32)
        u = ul[:uq, :].astype(jnp.bfloat16)
        y2 = ul[uq:, :]

        taps = []
        t = 0
        for ky in range(kk):
            for kx in range(kk):
                dy, dx = ky - pad, kx - pad
                if dy == 0 and dx == 0:
                    taps.append(u)
                else:
                    shift = (-(dy * W + dx)) % S
                    rolled = pltpu.roll(u, shift=shift, axis=1)
                    taps.append(rolled * mask_ref[t:t + 1, :])
                t += 1
        ucat = jnp.concatenate(taps, axis=0)

        y1 = jnp.dot(wg, ucat, preferred_element_type=jnp.float32) + bias

        m1 = jnp.mean(y1, axis=1, keepdims=True)
        m2 = jnp.mean(y2, axis=1, keepdims=True)
        mx = jnp.maximum(jnp.max(m1), jnp.max(m2))
        e1 = jnp.exp(m1 - mx)
        e2 = jnp.exp(m2 - mx)
        inv = 1.0 / (jnp.sum(e1) + jnp.sum(e2))
        o_ref[i] = (e1 * inv) * y1 + (e2 * inv) * y2


def kernel(x, wsq, wg, b_gwc, masks):
    N, C, H, W = x.shape
    S = H * W
    uq = wsq.shape[0] - C
    n_taps = masks.shape[0]
    kk = int(round(n_taps ** 0.5))
    kq = n_taps * uq

    nblk = 4 if N % 4 == 0 else 1
    G = N // nblk

    xr = x.reshape(N, C, S)

    body = functools.partial(_cru_body, uq, H, W, kk, nblk)

    est = (2 * nblk * C * S * 4 + 2 * nblk * C * S * 4 + kq * S * 2
           + 4 * C * S * 4 + n_taps * S * 2 + (uq + C) * C * 2
           + C * kq * 2 + C * 4)
    vmem_limit = int(min(max(2 * est, 32 * 1024 * 1024),
                         int(64 * 1024 * 1024 * 0.9)))

    out = pl.pallas_call(
        body,
        out_shape=jax.ShapeDtypeStruct((N, C, S), jnp.float32),
        grid=(G,),
        in_specs=[
            pl.BlockSpec((nblk, C, S), lambda b: (b, 0, 0)),
            pl.BlockSpec(wsq.shape, lambda b: (0, 0)),
            pl.BlockSpec(wg.shape, lambda b: (0, 0)),
            pl.BlockSpec(b_gwc.shape, lambda b: (0, 0)),
            pl.BlockSpec(masks.shape, lambda b: (0, 0)),
        ],
        out_specs=pl.BlockSpec((nblk, C, S), lambda b: (b, 0, 0)),
        compiler_params=pltpu.CompilerParams(
            dimension_semantics=("parallel",),
            vmem_limit_bytes=vmem_limit),
    )(xr, wsq, wg, b_gwc, masks)

    return out.reshape(N, C, H, W)
